# baseline (device time: 53679 ns/iter reference)
import os

import jax
import jax.numpy as jnp
from jax import lax
from jax.experimental import pallas as pl
from jax.experimental.pallas import tpu as pltpu

N_DEV = 4
F8_MAX = 448.0

_MODE = os.environ.get("KMODE", "full")


def kernel(x, w_mat):
    m_per, k = x.shape
    _, n = w_mat.shape
    n_per = n // N_DEV
    m_glob = N_DEV * m_per
    m_half = m_per // 2
    m_hold = m_per // 4
    m_top = m_per - m_hold

    def body(x_hbm_ref, w_hbm_ref, out_ref, x_vmem, w_buf, own_y, keep1,
             keep2, amax_ref, bf_send, bf_recv, bf2_send, bf2_recv,
             bf3_send, bf3_recv, f8h_send,
             f8h_recv, f8_send, f8_recv, x_sems, w_sems,
             bf_send_sem, bf_recv_sem, bf2_send_sem, bf2_recv_sem,
             bf3_send_sem, bf3_recv_sem,
             f8h_send_sem, f8h_recv_sem, f8_send_sems, f8_recv_sems,
             amax_send_sems, amax_recv_sems):
        my = lax.axis_index("i")

        def chunk_of(s):
            return (my + 1 + s) % N_DEV

        def x_dma(h):
            return pltpu.make_async_copy(
                x_hbm_ref.at[pl.ds(h * m_half, m_half), :],
                x_vmem.at[pl.ds(h * m_half, m_half), :],
                x_sems.at[h],
            )

        def w_dma(s):
            return pltpu.make_async_copy(
                w_hbm_ref.at[:, pl.ds(chunk_of(s) * n_per, n_per)],
                w_buf.at[s % 2],
                w_sems.at[s % 2],
            )

        def bf_rdma():
            return pltpu.make_async_remote_copy(
                src_ref=bf_send,
                dst_ref=bf_recv,
                send_sem=bf_send_sem,
                recv_sem=bf_recv_sem,
                device_id=(chunk_of(0),),
                device_id_type=pl.DeviceIdType.MESH,
            )

        def bf2_rdma():
            return pltpu.make_async_remote_copy(
                src_ref=bf2_send,
                dst_ref=bf2_recv,
                send_sem=bf2_send_sem,
                recv_sem=bf2_recv_sem,
                device_id=(chunk_of(1),),
                device_id_type=pl.DeviceIdType.MESH,
            )

        def bf3_rdma():
            return pltpu.make_async_remote_copy(
                src_ref=bf3_send,
                dst_ref=bf3_recv,
                send_sem=bf3_send_sem,
                recv_sem=bf3_recv_sem,
                device_id=(chunk_of(2),),
                device_id_type=pl.DeviceIdType.MESH,
            )

        def f8h_rdma():
            return pltpu.make_async_remote_copy(
                src_ref=f8h_send,
                dst_ref=f8h_recv,
                send_sem=f8h_send_sem,
                recv_sem=f8h_recv_sem,
                device_id=(chunk_of(1),),
                device_id_type=pl.DeviceIdType.MESH,
            )

        def f8_rdma():
            return pltpu.make_async_remote_copy(
                src_ref=f8_send,
                dst_ref=f8_recv,
                send_sem=f8_send_sems,
                recv_sem=f8_recv_sems,
                device_id=(chunk_of(2),),
                device_id_type=pl.DeviceIdType.MESH,
            )

        x_dma(0).start()
        w_dma(0).start()
        x_dma(1).start()
        w_dma(1).start()

        if _MODE != "gemm":
            with jax.named_scope("barrier_signal"):
                barrier_sem = pltpu.get_barrier_semaphore()
                for off in range(1, N_DEV):
                    pl.semaphore_signal(
                        barrier_sem, inc=1,
                        device_id=((my + off) % N_DEV,),
                        device_id_type=pl.DeviceIdType.MESH,
                    )

        am = jnp.float32(0.0)
        with jax.named_scope("gemm#s=0"):
            w_dma(0).wait()
            for h in range(2):
                x_dma(h).wait()
                yh = jnp.dot(
                    x_vmem[h * m_half:(h + 1) * m_half, :], w_buf[0],
                    preferred_element_type=jnp.float32,
                )
                yh = jnp.maximum(yh, 0.0)
                am = jnp.maximum(am, jnp.max(yh))
                bf_send[pl.ds(h * m_half, m_half), :] = yh.astype(
                    jnp.bfloat16
                )
            w_dma(2).start()
            if _MODE != "gemm":
                with jax.named_scope("barrier_wait"):
                    pl.semaphore_wait(barrier_sem, N_DEV - 1)
                bf_rdma().start()
        for s in range(1, N_DEV):
            with jax.named_scope(f"gemm#s={s}"):
                w_dma(s).wait()
                yj = jnp.dot(
                    x_vmem[...], w_buf[s % 2],
                    preferred_element_type=jnp.float32,
                )
                if s + 2 < N_DEV:
                    w_dma(s + 2).start()
                yj = jnp.maximum(yj, 0.0)
                am = jnp.maximum(am, jnp.max(yj))
                if s == 1:
                    bf2_send[...] = yj[:m_top, :].astype(jnp.bfloat16)
                    if _MODE != "gemm":
                        bf2_rdma().start()
                    keep1[...] = yj[m_top:, :]
                elif s == 2:
                    bf3_send[...] = yj[:m_top, :].astype(jnp.bfloat16)
                    if _MODE != "gemm":
                        bf3_rdma().start()
                    keep2[...] = yj[m_top:, :]
                else:
                    own_y[...] = yj
        amax_ref[0] = am * jnp.ones((8, 128), jnp.float32)

        amax_rdmas = []
        if _MODE != "gemm":
            with jax.named_scope("amax_exchange"):
                for off in range(1, N_DEV):
                    peer = (my + off) % N_DEV
                    r = pltpu.make_async_remote_copy(
                        src_ref=amax_ref.at[0],
                        dst_ref=amax_ref.at[N_DEV - off],
                        send_sem=amax_send_sems.at[off - 1],
                        recv_sem=amax_recv_sems.at[N_DEV - off - 1],
                        device_id=(peer,),
                        device_id_type=pl.DeviceIdType.MESH,
                    )
                    r.start()
                    amax_rdmas.append(r)
                for r in amax_rdmas:
                    r.wait_recv()
            gmax = jnp.max(amax_ref[...])
        else:
            gmax = jnp.max(amax_ref[0])
        inv_scale = F8_MAX / gmax
        scale = gmax / F8_MAX

        def quant(vals_f32):
            return jnp.clip(vals_f32 * inv_scale, 0.0, F8_MAX).astype(
                jnp.float8_e4m3fn
            )

        def epilogue(vals_f32):
            return quant(vals_f32).astype(jnp.float32) * scale

        if _MODE == "gemm":
            out_ref[pl.ds(chunk_of(0) * m_per, m_per), :] = epilogue(
                bf_send[...].astype(jnp.float32)
            )
            out_ref[pl.ds(chunk_of(1) * m_per, m_top), :] = epilogue(
                bf2_send[...].astype(jnp.float32)
            )
            out_ref[pl.ds(chunk_of(1) * m_per + m_top, m_hold), :] = (
                epilogue(keep1[...])
            )
            out_ref[pl.ds(chunk_of(2) * m_per, m_top), :] = epilogue(
                bf3_send[...].astype(jnp.float32)
            )
            out_ref[pl.ds(chunk_of(2) * m_per + m_top, m_hold), :] = (
                epilogue(keep2[...])
            )
            out_ref[pl.ds(my * m_per, m_per), :] = epilogue(own_y[...])
            return

        with jax.named_scope("f8_send"):
            f8h_send[...] = quant(keep1[...])
            f8h_rdma().start()
            f8_send[...] = quant(keep2[...])
            f8_rdma().start()

        with jax.named_scope("own_chunk"):
            out_ref[pl.ds(my * m_per, m_per), :] = epilogue(own_y[...])

        with jax.named_scope("bf_recv"):
            bf_rdma().wait_recv()
            origin3 = (my + 3) % N_DEV
            out_ref[pl.ds(origin3 * m_per, m_per), :] = epilogue(
                bf_recv[...].astype(jnp.float32)
            )

        with jax.named_scope("bf3_recv"):
            bf3_rdma().wait_recv()
            origin2f = (my + 1) % N_DEV
            out_ref[pl.ds(origin2f * m_per, m_top), :] = epilogue(
                bf3_recv[...].astype(jnp.float32)
            )

        with jax.named_scope("bf2_recv"):
            bf2_rdma().wait_recv()
            origin1 = (my + 2) % N_DEV
            out_ref[pl.ds(origin1 * m_per, m_top), :] = epilogue(
                bf2_recv[...].astype(jnp.float32)
            )

        with jax.named_scope("f8h_recv"):
            f8h_rdma().wait_recv()
            out_ref[pl.ds(origin1 * m_per + m_top, m_hold), :] = (
                f8h_recv[...].astype(jnp.float32) * scale
            )

        with jax.named_scope("f8_recv"):
            f8_rdma().wait_recv()
            origin2 = (my + 1) % N_DEV
            out_ref[pl.ds(origin2 * m_per + m_top, m_hold), :] = (
                f8_recv[...].astype(jnp.float32) * scale
            )

        with jax.named_scope("drain"):
            for r in amax_rdmas:
                r.wait_send()
            bf_rdma().wait_send()
            bf2_rdma().wait_send()
            bf3_rdma().wait_send()
            f8h_rdma().wait_send()
            f8_rdma().wait_send()

    return pl.pallas_call(
        body,
        out_shape=jax.ShapeDtypeStruct((m_glob, n_per), jnp.float32),
        in_specs=[
            pl.BlockSpec(memory_space=pl.ANY),
            pl.BlockSpec(memory_space=pl.ANY),
        ],
        out_specs=pl.BlockSpec(memory_space=pltpu.VMEM),
        scratch_shapes=[
            pltpu.VMEM((m_per, k), jnp.float32),
            pltpu.VMEM((2, k, n_per), jnp.float32),
            pltpu.VMEM((m_per, n_per), jnp.float32),
            pltpu.VMEM((m_hold, n_per), jnp.float32),
            pltpu.VMEM((m_hold, n_per), jnp.float32),
            pltpu.VMEM((N_DEV, 8, 128), jnp.float32),
            pltpu.VMEM((m_per, n_per), jnp.bfloat16),
            pltpu.VMEM((m_per, n_per), jnp.bfloat16),
            pltpu.VMEM((m_top, n_per), jnp.bfloat16),
            pltpu.VMEM((m_top, n_per), jnp.bfloat16),
            pltpu.VMEM((m_top, n_per), jnp.bfloat16),
            pltpu.VMEM((m_top, n_per), jnp.bfloat16),
            pltpu.VMEM((m_hold, n_per), jnp.float8_e4m3fn),
            pltpu.VMEM((m_hold, n_per), jnp.float8_e4m3fn),
            pltpu.VMEM((m_hold, n_per), jnp.float8_e4m3fn),
            pltpu.VMEM((m_hold, n_per), jnp.float8_e4m3fn),
            pltpu.SemaphoreType.DMA((2,)),
            pltpu.SemaphoreType.DMA((2,)),
            pltpu.SemaphoreType.DMA,
            pltpu.SemaphoreType.DMA,
            pltpu.SemaphoreType.DMA,
            pltpu.SemaphoreType.DMA,
            pltpu.SemaphoreType.DMA,
            pltpu.SemaphoreType.DMA,
            pltpu.SemaphoreType.DMA,
            pltpu.SemaphoreType.DMA,
            pltpu.SemaphoreType.DMA,
            pltpu.SemaphoreType.DMA,
            pltpu.SemaphoreType.DMA((3,)),
            pltpu.SemaphoreType.DMA((3,)),
        ],
        compiler_params=pltpu.CompilerParams(
            collective_id=None if _MODE == "gemm" else 0,
            vmem_limit_bytes=100 * 1024 * 1024,
        ),
    )(x, w_mat)


# device time: 52363 ns/iter; 1.0251x vs baseline; 1.0251x over previous
import os

import jax
import jax.numpy as jnp
from jax import lax
from jax.experimental import pallas as pl
from jax.experimental.pallas import tpu as pltpu

N_DEV = 4
F8_MAX = 448.0

_MODE = os.environ.get("KMODE", "full")


def kernel(x, w_mat):
    m_per, k = x.shape
    _, n = w_mat.shape
    n_per = n // N_DEV
    m_glob = N_DEV * m_per
    m_half = m_per // 2

    def body(x_hbm_ref, w_hbm_ref, out_ref, x_vmem, w_buf, own_y, keep1,
             keep2, amax_ref, bf_send, bf_recv, bf2_send, bf2_recv,
             bf3_send, bf3_recv, f8h_send,
             f8h_recv, f8_send, f8_recv, x_sems, w_sems,
             bf_send_sem, bf_recv_sem, bf2_send_sem, bf2_recv_sem,
             bf3_send_sem, bf3_recv_sem,
             f8h_send_sem, f8h_recv_sem, f8_send_sems, f8_recv_sems,
             amax_send_sems, amax_recv_sems):
        my = lax.axis_index("i")

        def chunk_of(s):
            return (my + 1 + s) % N_DEV

        def x_dma(h):
            return pltpu.make_async_copy(
                x_hbm_ref.at[pl.ds(h * m_half, m_half), :],
                x_vmem.at[pl.ds(h * m_half, m_half), :],
                x_sems.at[h],
            )

        def w_dma(s):
            return pltpu.make_async_copy(
                w_hbm_ref.at[:, pl.ds(chunk_of(s) * n_per, n_per)],
                w_buf.at[s % 2],
                w_sems.at[s % 2],
            )

        def bf_rdma():
            return pltpu.make_async_remote_copy(
                src_ref=bf_send,
                dst_ref=bf_recv,
                send_sem=bf_send_sem,
                recv_sem=bf_recv_sem,
                device_id=(chunk_of(0),),
                device_id_type=pl.DeviceIdType.MESH,
            )

        def bf2_rdma():
            return pltpu.make_async_remote_copy(
                src_ref=bf2_send,
                dst_ref=bf2_recv,
                send_sem=bf2_send_sem,
                recv_sem=bf2_recv_sem,
                device_id=(chunk_of(1),),
                device_id_type=pl.DeviceIdType.MESH,
            )

        def bf3_rdma():
            return pltpu.make_async_remote_copy(
                src_ref=bf3_send,
                dst_ref=bf3_recv,
                send_sem=bf3_send_sem,
                recv_sem=bf3_recv_sem,
                device_id=(chunk_of(2),),
                device_id_type=pl.DeviceIdType.MESH,
            )

        def f8h_rdma():
            return pltpu.make_async_remote_copy(
                src_ref=f8h_send,
                dst_ref=f8h_recv,
                send_sem=f8h_send_sem,
                recv_sem=f8h_recv_sem,
                device_id=(chunk_of(1),),
                device_id_type=pl.DeviceIdType.MESH,
            )

        def f8_rdma():
            return pltpu.make_async_remote_copy(
                src_ref=f8_send,
                dst_ref=f8_recv,
                send_sem=f8_send_sems,
                recv_sem=f8_recv_sems,
                device_id=(chunk_of(2),),
                device_id_type=pl.DeviceIdType.MESH,
            )

        x_dma(0).start()
        w_dma(0).start()
        x_dma(1).start()
        w_dma(1).start()

        if _MODE != "gemm":
            with jax.named_scope("barrier_signal"):
                barrier_sem = pltpu.get_barrier_semaphore()
                for off in range(1, N_DEV):
                    pl.semaphore_signal(
                        barrier_sem, inc=1,
                        device_id=((my + off) % N_DEV,),
                        device_id_type=pl.DeviceIdType.MESH,
                    )

        am = jnp.float32(0.0)
        with jax.named_scope("gemm#s=0"):
            w_dma(0).wait()
            for h in range(2):
                x_dma(h).wait()
                yh = jnp.dot(
                    x_vmem[h * m_half:(h + 1) * m_half, :], w_buf[0],
                    preferred_element_type=jnp.float32,
                )
                yh = jnp.maximum(yh, 0.0)
                am = jnp.maximum(am, jnp.max(yh))
                bf_send[pl.ds(h * m_half, m_half), :] = yh.astype(
                    jnp.bfloat16
                )
            w_dma(2).start()
            if _MODE != "gemm":
                with jax.named_scope("barrier_wait"):
                    pl.semaphore_wait(barrier_sem, N_DEV - 1)
                bf_rdma().start()
        for s in range(1, N_DEV):
            with jax.named_scope(f"gemm#s={s}"):
                w_dma(s).wait()
                yj = jnp.dot(
                    x_vmem[...], w_buf[s % 2],
                    preferred_element_type=jnp.float32,
                )
                if s + 2 < N_DEV:
                    w_dma(s + 2).start()
                yj = jnp.maximum(yj, 0.0)
                am = jnp.maximum(am, jnp.max(yj))
                if s == 1:
                    bf2_send[...] = yj[:m_half, :].astype(jnp.bfloat16)
                    if _MODE != "gemm":
                        bf2_rdma().start()
                    keep1[...] = yj[m_half:, :]
                elif s == 2:
                    bf3_send[...] = yj[:m_half, :].astype(jnp.bfloat16)
                    if _MODE != "gemm":
                        bf3_rdma().start()
                    keep2[...] = yj[m_half:, :]
                else:
                    own_y[...] = yj
        amax_ref[0] = am * jnp.ones((8, 128), jnp.float32)

        amax_rdmas = []
        if _MODE != "gemm":
            with jax.named_scope("amax_exchange"):
                for off in range(1, N_DEV):
                    peer = (my + off) % N_DEV
                    r = pltpu.make_async_remote_copy(
                        src_ref=amax_ref.at[0],
                        dst_ref=amax_ref.at[N_DEV - off],
                        send_sem=amax_send_sems.at[off - 1],
                        recv_sem=amax_recv_sems.at[N_DEV - off - 1],
                        device_id=(peer,),
                        device_id_type=pl.DeviceIdType.MESH,
                    )
                    r.start()
                    amax_rdmas.append(r)
                for r in amax_rdmas:
                    r.wait_recv()
            gmax = jnp.max(amax_ref[...])
        else:
            gmax = jnp.max(amax_ref[0])
        inv_scale = F8_MAX / gmax
        scale = gmax / F8_MAX

        def quant(vals_f32):
            return jnp.clip(vals_f32 * inv_scale, 0.0, F8_MAX).astype(
                jnp.float8_e4m3fn
            )

        def epilogue(vals_f32):
            return quant(vals_f32).astype(jnp.float32) * scale

        if _MODE == "gemm":
            out_ref[pl.ds(chunk_of(0) * m_per, m_per), :] = epilogue(
                bf_send[...].astype(jnp.float32)
            )
            out_ref[pl.ds(chunk_of(1) * m_per, m_half), :] = epilogue(
                bf2_send[...].astype(jnp.float32)
            )
            out_ref[pl.ds(chunk_of(1) * m_per + m_half, m_half), :] = (
                epilogue(keep1[...])
            )
            out_ref[pl.ds(chunk_of(2) * m_per, m_half), :] = epilogue(
                bf3_send[...].astype(jnp.float32)
            )
            out_ref[pl.ds(chunk_of(2) * m_per + m_half, m_half), :] = (
                epilogue(keep2[...])
            )
            out_ref[pl.ds(my * m_per, m_per), :] = epilogue(own_y[...])
            return

        with jax.named_scope("f8_send"):
            f8h_send[...] = quant(keep1[...])
            f8h_rdma().start()
            f8_send[...] = quant(keep2[...])
            f8_rdma().start()

        with jax.named_scope("own_chunk"):
            out_ref[pl.ds(my * m_per, m_per), :] = epilogue(own_y[...])

        with jax.named_scope("bf_recv"):
            bf_rdma().wait_recv()
            origin3 = (my + 3) % N_DEV
            out_ref[pl.ds(origin3 * m_per, m_per), :] = epilogue(
                bf_recv[...].astype(jnp.float32)
            )

        with jax.named_scope("bf3_recv"):
            bf3_rdma().wait_recv()
            origin2f = (my + 1) % N_DEV
            out_ref[pl.ds(origin2f * m_per, m_half), :] = epilogue(
                bf3_recv[...].astype(jnp.float32)
            )

        with jax.named_scope("bf2_recv"):
            bf2_rdma().wait_recv()
            origin1 = (my + 2) % N_DEV
            out_ref[pl.ds(origin1 * m_per, m_half), :] = epilogue(
                bf2_recv[...].astype(jnp.float32)
            )

        with jax.named_scope("f8h_recv"):
            f8h_rdma().wait_recv()
            out_ref[pl.ds(origin1 * m_per + m_half, m_half), :] = (
                f8h_recv[...].astype(jnp.float32) * scale
            )

        with jax.named_scope("f8_recv"):
            f8_rdma().wait_recv()
            origin2 = (my + 1) % N_DEV
            out_ref[pl.ds(origin2 * m_per + m_half, m_half), :] = (
                f8_recv[...].astype(jnp.float32) * scale
            )

        with jax.named_scope("drain"):
            for r in amax_rdmas:
                r.wait_send()
            bf_rdma().wait_send()
            bf2_rdma().wait_send()
            bf3_rdma().wait_send()
            f8h_rdma().wait_send()
            f8_rdma().wait_send()

    return pl.pallas_call(
        body,
        out_shape=jax.ShapeDtypeStruct((m_glob, n_per), jnp.float32),
        in_specs=[
            pl.BlockSpec(memory_space=pl.ANY),
            pl.BlockSpec(memory_space=pl.ANY),
        ],
        out_specs=pl.BlockSpec(memory_space=pltpu.VMEM),
        scratch_shapes=[
            pltpu.VMEM((m_per, k), jnp.float32),
            pltpu.VMEM((2, k, n_per), jnp.float32),
            pltpu.VMEM((m_per, n_per), jnp.float32),
            pltpu.VMEM((m_half, n_per), jnp.float32),
            pltpu.VMEM((m_half, n_per), jnp.float32),
            pltpu.VMEM((N_DEV, 8, 128), jnp.float32),
            pltpu.VMEM((m_per, n_per), jnp.bfloat16),
            pltpu.VMEM((m_per, n_per), jnp.bfloat16),
            pltpu.VMEM((m_half, n_per), jnp.bfloat16),
            pltpu.VMEM((m_half, n_per), jnp.bfloat16),
            pltpu.VMEM((m_half, n_per), jnp.bfloat16),
            pltpu.VMEM((m_half, n_per), jnp.bfloat16),
            pltpu.VMEM((m_half, n_per), jnp.float8_e4m3fn),
            pltpu.VMEM((m_half, n_per), jnp.float8_e4m3fn),
            pltpu.VMEM((m_half, n_per), jnp.float8_e4m3fn),
            pltpu.VMEM((m_half, n_per), jnp.float8_e4m3fn),
            pltpu.SemaphoreType.DMA((2,)),
            pltpu.SemaphoreType.DMA((2,)),
            pltpu.SemaphoreType.DMA,
            pltpu.SemaphoreType.DMA,
            pltpu.SemaphoreType.DMA,
            pltpu.SemaphoreType.DMA,
            pltpu.SemaphoreType.DMA,
            pltpu.SemaphoreType.DMA,
            pltpu.SemaphoreType.DMA,
            pltpu.SemaphoreType.DMA,
            pltpu.SemaphoreType.DMA,
            pltpu.SemaphoreType.DMA,
            pltpu.SemaphoreType.DMA((3,)),
            pltpu.SemaphoreType.DMA((3,)),
        ],
        compiler_params=pltpu.CompilerParams(
            collective_id=None if _MODE == "gemm" else 0,
            vmem_limit_bytes=100 * 1024 * 1024,
        ),
    )(x, w_mat)
